# baseline (speedup 1.0000x reference)
import functools

import numpy as np
import jax
import jax.numpy as jnp
from jax import lax
from jax.experimental import pallas as pl
from jax.experimental.pallas import tpu as pltpu
from jax.experimental.pallas import tpu_sc as plsc

PROB = 0.25
B, T, C, D = 16, 2048, 4, 128
NC, NS = 2, 16
HALF = T // 2
ZCH = 128


@functools.lru_cache(maxsize=None)
def _drop_indices(b: int):
    cpu = jax.devices("cpu")[0]
    with jax.default_device(cpu):
        perm = np.asarray(jax.random.permutation(jax.random.key(1), b))
    n = 1 if b == 1 else int(b * PROB)
    return tuple(int(i) for i in perm[:n])


def _make_sc_kernel(drop):
    mesh = plsc.VectorSubcoreMesh(core_axis_name="c", subcore_axis_name="s")

    @functools.partial(
        pl.kernel,
        out_type=jax.ShapeDtypeStruct((B, T, C, D), jnp.float32),
        mesh=mesh,
        scratch_types=[
            pltpu.VMEM((ZCH, C, D), jnp.float32),
            pltpu.VMEM((1, C, D), jnp.float32),
        ],
    )
    def _sc_dropout(emb0_hbm, out_hbm, zbuf, lbuf):
        wid = lax.axis_index("s") * NC + lax.axis_index("c")
        r = wid // 2
        h = wid % 2
        t0 = h * HALF
        dropped = functools.reduce(
            jnp.logical_or, [r == di for di in drop])

        @pl.when(jnp.logical_not(dropped))
        def _copy():
            pltpu.sync_copy(emb0_hbm.at[r, pl.ds(t0, HALF)],
                            out_hbm.at[r, pl.ds(t0, HALF)])

        @pl.when(dropped)
        def _zero():
            zero16 = jnp.zeros((16,), jnp.float32)

            def body(i, carry):
                for j in range(C):
                    for k in range(D // 16):
                        zbuf[i, j, pl.ds(k * 16, 16)] = zero16
                return carry

            lax.fori_loop(0, ZCH, body, 0)
            for q in range(HALF // ZCH):
                pltpu.sync_copy(zbuf,
                                out_hbm.at[r, pl.ds(t0 + q * ZCH, ZCH)])

            @pl.when(h == 1)
            def _fix_last():
                pltpu.sync_copy(emb0_hbm.at[r, pl.ds(T - 1, 1)], lbuf)
                pltpu.sync_copy(lbuf, out_hbm.at[r, pl.ds(T - 1, 1)])

    return _sc_dropout


@functools.partial(jax.jit, static_argnums=(1,))
def _run(emb0, drop):
    return _make_sc_kernel(drop)(emb0)


_drop_indices(16)


def kernel(emb0):
    return _run(emb0, _drop_indices(emb0.shape[0]))


# speedup vs baseline: 1.0005x; 1.0005x over previous
import functools

import numpy as np
import jax
import jax.numpy as jnp
from jax import lax
from jax.experimental import pallas as pl
from jax.experimental.pallas import tpu as pltpu
from jax.experimental.pallas import tpu_sc as plsc

PROB = 0.25
B, T, C, D = 16, 2048, 4, 128
NC, NS = 2, 16
HALF = T // 2
ZCH = 128


@functools.lru_cache(maxsize=None)
def _drop_indices(b: int):
    cpu = jax.devices("cpu")[0]
    with jax.default_device(cpu):
        perm = np.asarray(jax.random.permutation(jax.random.key(1), b))
    n = 1 if b == 1 else int(b * PROB)
    return tuple(int(i) for i in perm[:n])


def _make_sc_kernel(drop):
    mesh = plsc.VectorSubcoreMesh(core_axis_name="c", subcore_axis_name="s")

    @functools.partial(
        pl.kernel,
        out_type=jax.ShapeDtypeStruct((B, T, C, D), jnp.float32),
        mesh=mesh,
        scratch_types=[
            pltpu.VMEM((ZCH, C, D), jnp.float32),
            pltpu.VMEM((1, C, D), jnp.float32),
            pltpu.SemaphoreType.DMA,
        ],
    )
    def _sc_dropout(emb0_hbm, out_hbm, zbuf, lbuf, sem):
        wid = lax.axis_index("s") * NC + lax.axis_index("c")
        r = wid // 2
        h = wid % 2
        t0 = h * HALF
        dropped = functools.reduce(
            jnp.logical_or, [r == di for di in drop])

        @pl.when(jnp.logical_not(dropped))
        def _copy():
            dmas = [
                pltpu.async_copy(
                    emb0_hbm.at[r, pl.ds(t0 + q * ZCH, ZCH)],
                    out_hbm.at[r, pl.ds(t0 + q * ZCH, ZCH)], sem)
                for q in range(HALF // ZCH)
            ]
            for dma in dmas:
                dma.wait()

        @pl.when(dropped)
        def _zero():
            zero16 = jnp.zeros((16,), jnp.float32)

            def body(i, carry):
                for j in range(C):
                    for k in range(D // 16):
                        zbuf[i, j, pl.ds(k * 16, 16)] = zero16
                return carry

            lax.fori_loop(0, ZCH, body, 0)
            dmas = [
                pltpu.async_copy(
                    zbuf, out_hbm.at[r, pl.ds(t0 + q * ZCH, ZCH)], sem)
                for q in range(HALF // ZCH)
            ]
            for dma in dmas:
                dma.wait()

            @pl.when(h == 1)
            def _fix_last():
                pltpu.sync_copy(emb0_hbm.at[r, pl.ds(T - 1, 1)], lbuf)
                pltpu.sync_copy(lbuf, out_hbm.at[r, pl.ds(T - 1, 1)])

    return _sc_dropout


@functools.partial(jax.jit, static_argnums=(1,))
def _run(emb0, drop):
    return _make_sc_kernel(drop)(emb0)


_drop_indices(16)


def kernel(emb0):
    return _run(emb0, _drop_indices(emb0.shape[0]))


# speedup vs baseline: 36.2745x; 36.2579x over previous
import functools

import numpy as np
import jax
import jax.numpy as jnp
from jax.experimental import pallas as pl
from jax.experimental.pallas import tpu as pltpu

PROB = 0.25
LH = 8


@functools.lru_cache(maxsize=None)
def _drop_indices(b: int):
    cpu = jax.devices("cpu")[0]
    with jax.default_device(cpu):
        perm = np.asarray(jax.random.permutation(jax.random.key(1), b))
    n = 1 if b == 1 else int(b * PROB)
    return tuple(int(i) for i in perm[:n])


def _order_tables(b, drop):
    kept = [i for i in range(b) if i not in drop]
    nd = len(drop)
    pos_d = set(1 + j * (b // nd) for j in range(nd))
    order, ki, di = [], 0, 0
    for pos in range(b):
        if pos in pos_d:
            order.append(sorted(drop)[di])
            di += 1
        else:
            order.append(kept[ki])
            ki += 1
    read_row = []
    for pos in range(b):
        if pos in pos_d:
            read_row.append(read_row[pos - 1])
        else:
            read_row.append(order[pos])
    return tuple(order), tuple(read_row), tuple(sorted(pos_d))


def _static_lookup(i, table):
    p = table[0]
    for idx in range(1, len(table)):
        p = jnp.where(i == idx, table[idx], p)
    return p


def _masked_copy_kernel(x_ref, last_ref, o_ref, *, pos_d, t):
    i = pl.program_id(0)
    dropped = functools.reduce(jnp.logical_or, [i == p for p in pos_d])

    @pl.when(~dropped)
    def _copy():
        o_ref[...] = x_ref[...]

    @pl.when(dropped)
    def _zero():
        last = last_ref[0, LH - 1, :, :]
        tids = jax.lax.broadcasted_iota(jnp.int32, o_ref.shape, 1)
        o_ref[...] = jnp.where(tids == t - 1, last[None, None], 0.0)


@functools.partial(jax.jit, static_argnums=(1,))
def _run(emb0, drop):
    b, t, c, d = emb0.shape
    order, read_row, pos_d = _order_tables(b, drop)

    return pl.pallas_call(
        functools.partial(_masked_copy_kernel, pos_d=pos_d, t=t),
        grid=(b,),
        in_specs=[
            pl.BlockSpec((1, t, c, d),
                         lambda i: (_static_lookup(i, read_row), 0, 0, 0)),
            pl.BlockSpec((1, LH, c, d),
                         lambda i: (_static_lookup(i, order),
                                    t // LH - 1, 0, 0)),
        ],
        out_specs=pl.BlockSpec((1, t, c, d),
                               lambda i: (_static_lookup(i, order), 0, 0, 0)),
        out_shape=jax.ShapeDtypeStruct((b, t, c, d), emb0.dtype),
        compiler_params=pltpu.CompilerParams(
            dimension_semantics=("parallel",)),
    )(emb0, emb0)


_drop_indices(16)


def kernel(emb0):
    return _run(emb0, _drop_indices(emb0.shape[0]))


# speedup vs baseline: 39.3342x; 1.0843x over previous
import functools

import numpy as np
import jax
import jax.numpy as jnp
from jax.experimental import pallas as pl
from jax.experimental.pallas import tpu as pltpu

PROB = 0.25
LH = 8


@functools.lru_cache(maxsize=None)
def _drop_indices(b: int):
    cpu = jax.devices("cpu")[0]
    with jax.default_device(cpu):
        perm = np.asarray(jax.random.permutation(jax.random.key(1), b))
    n = 1 if b == 1 else int(b * PROB)
    return tuple(int(i) for i in perm[:n])


def _prev_kept_table(b, drop):
    tab, prev = [], None
    for i in range(b):
        if i not in drop:
            prev = i
        tab.append(prev)
    first_kept = next(i for i in range(b) if i not in drop)
    return tuple(first_kept if v is None else v for v in tab)


def _masked_copy_kernel(x_ref, last_ref, o_ref, *, drop, t):
    i = pl.program_id(0)
    dropped = functools.reduce(jnp.logical_or, [i == di for di in drop])

    @pl.when(~dropped)
    def _copy():
        o_ref[...] = x_ref[...]

    @pl.when(dropped)
    def _zero():
        last = last_ref[0, LH - 1, :, :]
        tids = jax.lax.broadcasted_iota(jnp.int32, o_ref.shape, 1)
        o_ref[...] = jnp.where(tids == t - 1, last[None, None], 0.0)


@functools.partial(jax.jit, static_argnums=(1,))
def _run(emb0, drop):
    b, t, c, d = emb0.shape
    prev_kept = _prev_kept_table(b, drop)

    def in_map(i):
        p = i
        for di in drop:
            p = jnp.where(i == di, prev_kept[di], p)
        return (p, 0, 0, 0)

    return pl.pallas_call(
        functools.partial(_masked_copy_kernel, drop=drop, t=t),
        grid=(b,),
        in_specs=[
            pl.BlockSpec((1, t, c, d), in_map),
            pl.BlockSpec((1, LH, c, d), lambda i: (i, t // LH - 1, 0, 0)),
        ],
        out_specs=pl.BlockSpec((1, t, c, d), lambda i: (i, 0, 0, 0)),
        out_shape=jax.ShapeDtypeStruct((b, t, c, d), emb0.dtype),
        compiler_params=pltpu.CompilerParams(
            dimension_semantics=("parallel",)),
    )(emb0, emb0)


_drop_indices(16)


def kernel(emb0):
    return _run(emb0, _drop_indices(emb0.shape[0]))


# speedup vs baseline: 39.4518x; 1.0030x over previous
import functools

import numpy as np
import jax
import jax.numpy as jnp
from jax.experimental import pallas as pl
from jax.experimental.pallas import tpu as pltpu

PROB = 0.25
LH = 8


@functools.lru_cache(maxsize=None)
def _drop_indices(b: int):
    cpu = jax.devices("cpu")[0]
    with jax.default_device(cpu):
        perm = np.asarray(jax.random.permutation(jax.random.key(1), b))
    n = 1 if b == 1 else int(b * PROB)
    return tuple(int(i) for i in perm[:n])


def _prev_kept_table(b, drop):
    tab, prev = [], None
    for i in range(b):
        if i not in drop:
            prev = i
        tab.append(prev)
    first_kept = next(i for i in range(b) if i not in drop)
    return tuple(first_kept if v is None else v for v in tab)


def _masked_copy_kernel(x_ref, last_ref, o_ref, *, drop, t):
    i = pl.program_id(0)
    dropped = functools.reduce(jnp.logical_or, [i == di for di in drop])

    @pl.when(~dropped)
    def _copy():
        o_ref[...] = x_ref[...]

    @pl.when(dropped)
    def _zero():
        last = last_ref[0, LH - 1, :, :]
        tids = jax.lax.broadcasted_iota(jnp.int32, o_ref.shape, 1)
        o_ref[...] = jnp.where(tids == t - 1, last[None, None], 0.0)


@functools.partial(jax.jit, static_argnums=(1,))
def _run(emb0, drop):
    b, t, c, d = emb0.shape
    prev_kept = _prev_kept_table(b, drop)

    def in_map(i):
        p = i
        for di in drop:
            p = jnp.where(i == di, prev_kept[di], p)
        return (p, 0, 0, 0)

    return pl.pallas_call(
        functools.partial(_masked_copy_kernel, drop=drop, t=t),
        grid=(b,),
        in_specs=[
            pl.BlockSpec((1, t, c, d), in_map),
            pl.BlockSpec((1, LH, c, d), lambda i: (i, t // LH - 1, 0, 0)),
        ],
        out_specs=pl.BlockSpec((1, t, c, d), lambda i: (i, 0, 0, 0)),
        out_shape=jax.ShapeDtypeStruct((b, t, c, d), emb0.dtype),
        compiler_params=pltpu.CompilerParams(
            dimension_semantics=("arbitrary",)),
    )(emb0, emb0)


_drop_indices(16)


def kernel(emb0):
    return _run(emb0, _drop_indices(emb0.shape[0]))
